# Initial kernel scaffold; baseline (speedup 1.0000x reference)
import jax
import jax.numpy as jnp
from jax.experimental import pallas as pl


def kernel(x, lens, table, W, b):
    raise NotImplementedError("write your pallas kernel here")


# speedup vs baseline: 29.9768x; 29.9768x over previous
import functools

import jax
import jax.numpy as jnp
from jax import lax
from jax.experimental import pallas as pl
from jax.experimental.pallas import tpu as pltpu
from jax.experimental.pallas import tpu_sc as plsc

V = 100000
D = 128
L_SEQ = 200
B = 4096

_PBLK = 8192


def _proj_body(b_ref, t_ref, w_ref, q_ref):
    t = t_ref[...]
    w = w_ref[...]
    q = lax.dot_general(t, w, (((1,), (1,)), ((), ())),
                        preferred_element_type=jnp.float32)
    q_ref[...] = q * (1.0 / L_SEQ) + b_ref[0] * (1.0 / L_SEQ)


def _project_table(table, W, b):
    grid = (V + _PBLK - 1) // _PBLK
    q2 = pl.pallas_call(
        _proj_body,
        grid=(grid,),
        in_specs=[
            pl.BlockSpec(memory_space=pltpu.SMEM),
            pl.BlockSpec((_PBLK, D), lambda i: (i, 0)),
            pl.BlockSpec((1, D), lambda i: (0, 0)),
        ],
        out_specs=pl.BlockSpec((_PBLK, 1), lambda i: (i, 0)),
        out_shape=jax.ShapeDtypeStruct((V, 1), jnp.float32),
    )(b, table, W)
    return q2.reshape(V)


_NC, _NS = 2, 16
_NW = _NC * _NS
_BPW = B // _NW
_G = _BPW // 16

@functools.cache
def _sc_gather_sum_fn():
    mesh = plsc.VectorSubcoreMesh(core_axis_name="c", subcore_axis_name="s")

    @functools.partial(
        pl.kernel,
        mesh=mesh,
        compiler_params=pltpu.CompilerParams(needs_layout_passes=False),
        out_type=jax.ShapeDtypeStruct((B,), jnp.float32),
        scratch_types=[
            pltpu.VMEM((V,), jnp.float32),
            pltpu.VMEM((L_SEQ, _BPW), jnp.int32),
            pltpu.VMEM((_BPW,), jnp.float32),
        ],
    )
    def _sc_gather_sum(q_hbm, x_hbm, out_hbm, q_v, x_v, acc_v):
        wid = lax.axis_index("s") * _NC + lax.axis_index("c")
        base = wid * _BPW
        pltpu.sync_copy(q_hbm, q_v)
        pltpu.sync_copy(x_hbm.at[:, pl.ds(base, _BPW)], x_v)
        for g in range(_G):
            def body(l, acc, g=g):
                idx = x_v[l, pl.ds(g * 16, 16)]
                return acc + plsc.load_gather(q_v, [idx])
            acc = lax.fori_loop(0, L_SEQ, body, jnp.zeros((16,), jnp.float32))
            acc_v[pl.ds(g * 16, 16)] = acc
        pltpu.sync_copy(acc_v, out_hbm.at[pl.ds(base, _BPW)])

    return _sc_gather_sum


def kernel(x, lens, table, W, b):
    del lens
    q = _project_table(table, W, b)
    return _sc_gather_sum_fn()(q, x)


# speedup vs baseline: 50.1168x; 1.6719x over previous
import functools

import jax
import jax.numpy as jnp
from jax import lax
from jax.experimental import pallas as pl
from jax.experimental.pallas import tpu as pltpu
from jax.experimental.pallas import tpu_sc as plsc

V = 100000
D = 128
L_SEQ = 200
B = 4096
_INV_L = 1.0 / L_SEQ

_QROWS = 64
_PBLK = _QROWS * D
_GRID = (V + _PBLK - 1) // _PBLK
_QR_TOTAL = _GRID * _QROWS


def _proj_body(b_ref, t_ref, w_ref, q_ref):
    t = t_ref[...]
    w = w_ref[...]
    s = (t * w).reshape(_QROWS, D, D)
    q_ref[...] = jnp.sum(s, axis=2) * _INV_L + b_ref[0] * _INV_L


def _project_table(table, W, b):
    q2 = pl.pallas_call(
        _proj_body,
        grid=(_GRID,),
        in_specs=[
            pl.BlockSpec(memory_space=pltpu.SMEM),
            pl.BlockSpec((_PBLK, D), lambda i: (i, 0)),
            pl.BlockSpec((1, D), lambda i: (0, 0)),
        ],
        out_specs=pl.BlockSpec((_QROWS, D), lambda i: (i, 0)),
        out_shape=jax.ShapeDtypeStruct((_QR_TOTAL, D), jnp.float32),
    )(b, table, W)
    return q2.reshape(_QR_TOTAL * D)


_NC, _NS = 2, 16
_NW = _NC * _NS
_BPW = B // _NW
_G = _BPW // 16
_UNROLL = 8


@functools.cache
def _sc_gather_sum_fn():
    mesh = plsc.VectorSubcoreMesh(core_axis_name="c", subcore_axis_name="s")

    @functools.partial(
        pl.kernel,
        mesh=mesh,
        compiler_params=pltpu.CompilerParams(needs_layout_passes=False),
        out_type=jax.ShapeDtypeStruct((B,), jnp.float32),
        scratch_types=[
            pltpu.VMEM((V,), jnp.float32),
            pltpu.VMEM((L_SEQ, _BPW), jnp.int32),
            pltpu.VMEM((_BPW,), jnp.float32),
            pltpu.SemaphoreType.DMA,
            pltpu.SemaphoreType.DMA,
        ],
    )
    def _sc_gather_sum(q_hbm, x_hbm, out_hbm, q_v, x_v, acc_v, sem_q, sem_x):
        wid = lax.axis_index("s") * _NC + lax.axis_index("c")
        base = wid * _BPW
        cq = pltpu.async_copy(q_hbm.at[pl.ds(0, V)], q_v, sem_q)
        cx = pltpu.async_copy(x_hbm.at[:, pl.ds(base, _BPW)], x_v, sem_x)
        cx.wait()
        cq.wait()
        for g in range(_G):
            def body(i, acc, g=g):
                l0 = i * _UNROLL
                for u in range(_UNROLL):
                    idx = x_v[l0 + u, pl.ds(g * 16, 16)]
                    acc = acc + plsc.load_gather(q_v, [idx])
                return acc
            acc = lax.fori_loop(0, L_SEQ // _UNROLL, body,
                                jnp.zeros((16,), jnp.float32))
            acc_v[pl.ds(g * 16, 16)] = acc
        pltpu.sync_copy(acc_v, out_hbm.at[pl.ds(base, _BPW)])

    return _sc_gather_sum


def kernel(x, lens, table, W, b):
    del lens
    q = _project_table(table, W, b)
    return _sc_gather_sum_fn()(q, x)


# speedup vs baseline: 52.3722x; 1.0450x over previous
import functools

import jax
import jax.numpy as jnp
from jax import lax
from jax.experimental import pallas as pl
from jax.experimental.pallas import tpu as pltpu
from jax.experimental.pallas import tpu_sc as plsc

V = 100000
D = 128
L_SEQ = 200
B = 4096
_INV_L = 1.0 / L_SEQ

_QROWS = 128
_PBLK = _QROWS * D
_GRID = (V + _PBLK - 1) // _PBLK
_QR_TOTAL = _GRID * _QROWS


def _proj_body(b_ref, t_ref, w_ref, q_ref):
    t = t_ref[...]
    w = w_ref[...]
    s = (t * w).reshape(_QROWS, D, D)
    q_ref[...] = jnp.sum(s, axis=2) * _INV_L + b_ref[0] * _INV_L


def _project_table(table, W, b):
    q2 = pl.pallas_call(
        _proj_body,
        grid=(_GRID,),
        in_specs=[
            pl.BlockSpec(memory_space=pltpu.SMEM),
            pl.BlockSpec((_PBLK, D), lambda i: (i, 0)),
            pl.BlockSpec((1, D), lambda i: (0, 0)),
        ],
        out_specs=pl.BlockSpec((_QROWS, D), lambda i: (i, 0)),
        out_shape=jax.ShapeDtypeStruct((_QR_TOTAL, D), jnp.float32),
    )(b, table, W)
    return q2.reshape(_QR_TOTAL * D)


_NC, _NS = 2, 16
_NW = _NC * _NS
_BPW = B // _NW
_G = _BPW // 16


@functools.cache
def _sc_gather_sum_fn():
    mesh = plsc.VectorSubcoreMesh(core_axis_name="c", subcore_axis_name="s")

    @functools.partial(
        pl.kernel,
        mesh=mesh,
        compiler_params=pltpu.CompilerParams(needs_layout_passes=False),
        out_type=jax.ShapeDtypeStruct((B,), jnp.float32),
        scratch_types=[
            pltpu.VMEM((V,), jnp.float32),
            pltpu.VMEM((L_SEQ, _BPW), jnp.int32),
            pltpu.VMEM((_BPW,), jnp.float32),
            pltpu.SemaphoreType.DMA,
            pltpu.SemaphoreType.DMA,
        ],
    )
    def _sc_gather_sum(q_hbm, x_hbm, out_hbm, q_v, x_v, acc_v, sem_q, sem_x):
        wid = lax.axis_index("s") * _NC + lax.axis_index("c")
        base = wid * _BPW
        cq = pltpu.async_copy(q_hbm.at[pl.ds(0, V)], q_v, sem_q)
        cx = pltpu.async_copy(x_hbm.at[:, pl.ds(base, _BPW)], x_v, sem_x)
        cx.wait()
        cq.wait()

        def body(l, accs):
            new = []
            for g in range(_G):
                idx = x_v[l, pl.ds(g * 16, 16)]
                new.append(accs[g] + plsc.load_gather(q_v, [idx]))
            return tuple(new)

        accs = lax.fori_loop(
            0, L_SEQ, body,
            tuple(jnp.zeros((16,), jnp.float32) for _ in range(_G)))
        for g in range(_G):
            acc_v[pl.ds(g * 16, 16)] = accs[g]
        pltpu.sync_copy(acc_v, out_hbm.at[pl.ds(base, _BPW)])

    return _sc_gather_sum


def kernel(x, lens, table, W, b):
    del lens
    q = _project_table(table, W, b)
    return _sc_gather_sum_fn()(q, x)
